# baseline (speedup 1.0000x reference)
import functools
import math

import jax
import jax.numpy as jnp
from jax import lax
from jax.experimental import pallas as pl
from jax.experimental.pallas import tpu as pltpu
from jax.experimental.pallas import tpu_sc as plsc

_VOCAB = 1000000
_D = 64
_SCALE = math.sqrt(_D)

_INFO = plsc.get_sparse_core_info()
_NC = _INFO.num_cores
_NS = _INFO.num_subcores
_NW = _NC * _NS
_CHUNK = 128
_LANES = 16


def _make_kernel(n_chunks):
    mesh = plsc.VectorSubcoreMesh(core_axis_name="c", subcore_axis_name="s")

    @functools.partial(
        pl.kernel,
        mesh=mesh,
        out_type=jax.ShapeDtypeStruct((_NW, n_chunks, _CHUNK, _D), jnp.float32),
        scratch_types=[
            pltpu.VMEM((n_chunks, _CHUNK), jnp.int32),
            pltpu.VMEM((_CHUNK, _D), jnp.float32),
            pltpu.SemaphoreType.DMA,
        ],
        compiler_params=pltpu.CompilerParams(use_tc_tiling_on_sc=False),
    )
    def k(x_hbm, table_hbm, out_hbm, idx_v, rows_v, gsem):
        wid = lax.axis_index("s") * _NC + lax.axis_index("c")
        pltpu.sync_copy(x_hbm.at[wid], idx_v)

        def scale_row(r, _):
            for c in range(_D // _LANES):
                sl = pl.ds(c * _LANES, _LANES)
                rows_v[r, sl] = rows_v[r, sl] * _SCALE
            return 0

        def step(g, _):
            pltpu.async_copy(table_hbm.at[idx_v.at[g]], rows_v, gsem).wait()
            lax.fori_loop(0, _CHUNK, scale_row, 0)
            pltpu.sync_copy(rows_v, out_hbm.at[wid, g])
            return 0

        lax.fori_loop(0, n_chunks, step, 0)

    return k


def kernel(x, table):
    b, s = x.shape
    total = b * s
    n_chunks = total // (_NW * _CHUNK)
    xr = x.reshape(_NW, n_chunks, _CHUNK).astype(jnp.int32)
    out = _make_kernel(n_chunks)(xr, table)
    return out.reshape(b, s, _D)


# speedup vs baseline: 1.1873x; 1.1873x over previous
import functools
import math

import jax
import jax.numpy as jnp
from jax import lax
from jax.experimental import pallas as pl
from jax.experimental.pallas import tpu as pltpu
from jax.experimental.pallas import tpu_sc as plsc

_D = 64
_SCALE = math.sqrt(_D)

_INFO = plsc.get_sparse_core_info()
_NC = _INFO.num_cores
_NS = _INFO.num_subcores
_NW = _NC * _NS
_CHUNK = 128
_LANES = 16
_NBUF = 4
_LA = 2


def _make_kernel(n_chunks):
    mesh = plsc.VectorSubcoreMesh(core_axis_name="c", subcore_axis_name="s")

    @functools.partial(
        pl.kernel,
        mesh=mesh,
        out_type=jax.ShapeDtypeStruct((_NW, n_chunks, _CHUNK, _D), jnp.float32),
        scratch_types=[
            pltpu.VMEM((n_chunks, _CHUNK), jnp.int32),
            pltpu.VMEM((_NBUF, _CHUNK, _D), jnp.float32),
            [pltpu.SemaphoreType.DMA] * _NBUF,
            [pltpu.SemaphoreType.DMA] * _NBUF,
        ],
        compiler_params=pltpu.CompilerParams(use_tc_tiling_on_sc=False),
    )
    def k(x_hbm, table_hbm, out_hbm, idx_v, rows_v, gsems, ssems):
        wid = lax.axis_index("s") * _NC + lax.axis_index("c")
        pltpu.sync_copy(x_hbm.at[wid], idx_v)

        for g in range(_LA):
            pltpu.async_copy(
                table_hbm.at[idx_v.at[g]], rows_v.at[g % _NBUF], gsems[g % _NBUF]
            )

        def outer(o, _):
            for b in range(_NBUF):
                g = o * _NBUF + b
                pltpu.make_async_copy(
                    table_hbm.at[idx_v.at[g]], rows_v.at[b], gsems[b]
                ).wait()

                @plsc.parallel_loop(0, _CHUNK, step=1, unroll=4)
                def _scale(r):
                    for c in range(_D // _LANES):
                        sl = pl.ds(c * _LANES, _LANES)
                        rows_v[b, r, sl] = rows_v[b, r, sl] * _SCALE

                pltpu.async_copy(rows_v.at[b], out_hbm.at[wid, g], ssems[b])

                nb = (b + _LA) % _NBUF

                @pl.when(g + _LA < n_chunks)
                def _():
                    @pl.when(g >= _LA)
                    def _():
                        pltpu.make_async_copy(
                            rows_v.at[nb], out_hbm.at[wid, 0], ssems[nb]
                        ).wait()

                    pltpu.async_copy(
                        table_hbm.at[idx_v.at[g + _LA]], rows_v.at[nb], gsems[nb]
                    )

            return 0

        lax.fori_loop(0, n_chunks // _NBUF, outer, 0)

        for b in range(_NBUF):
            pltpu.make_async_copy(rows_v.at[b], out_hbm.at[wid, 0], ssems[b]).wait()

    return k


def kernel(x, table):
    b, s = x.shape
    total = b * s
    n_chunks = total // (_NW * _CHUNK)
    xr = x.reshape(_NW, n_chunks, _CHUNK).astype(jnp.int32)
    out = _make_kernel(n_chunks)(xr, table)
    return out.reshape(b, s, _D)
